# baseline (speedup 1.0000x reference)
import jax
import jax.numpy as jnp
from jax import lax
from jax.experimental import pallas as pl
from jax.experimental.pallas import tpu as pltpu
from jax.experimental.pallas import tpu_sc as plsc

_F_OUT = 64

_NS = 16
_MASK_N = 512000
_ZCHUNK = 8000
_ZCOPIES = (_MASK_N // _NS) // _ZCHUNK
_LBL_COLS = 128
_LBL_ROWS = 25
_LBL_PAD = _NS * _LBL_ROWS * _LBL_COLS

_BLK = 4000


def _sc_mask_body(labels_ref, mask_ref, zeros_v, idx_v, ones_v, sem):
    c = lax.axis_index("c")
    s = lax.axis_index("s")

    @pl.when(c == 0)
    def _zero_phase():
        def _fz(i, carry):
            zeros_v[pl.ds(i * 16, 16)] = jnp.zeros((16,), jnp.float32)
            return carry
        lax.fori_loop(0, _ZCHUNK // 16, _fz, 0)
        for j in range(_LBL_COLS // 16):
            ones_v[pl.ds(j * 16, 16)] = jnp.ones((16,), jnp.float32)
        base = s * (_MASK_N // _NS)
        for k in range(_ZCOPIES):
            pltpu.sync_copy(zeros_v,
                            mask_ref.at[pl.ds(base + k * _ZCHUNK, _ZCHUNK)])

    plsc.subcore_barrier()

    @pl.when(c == 0)
    def _scatter_phase():
        pltpu.sync_copy(labels_ref.at[s], idx_v)
        copies = [
            pltpu.async_copy(ones_v, mask_ref.at[idx_v.at[j]], sem)
            for j in range(_LBL_ROWS)
        ]
        for cp in copies:
            cp.wait()


def _make_mask(labels):
    lbl = labels.astype(jnp.int32)
    pad = _LBL_PAD - lbl.shape[0]
    lbl = jnp.concatenate([lbl, jnp.broadcast_to(lbl[-1:], (pad,))])
    lbl3 = lbl.reshape(_NS, _LBL_ROWS, _LBL_COLS)
    return pl.kernel(
        _sc_mask_body,
        out_type=jax.ShapeDtypeStruct((_MASK_N,), jnp.float32),
        mesh=plsc.VectorSubcoreMesh(core_axis_name="c", subcore_axis_name="s"),
        scratch_types=[
            pltpu.VMEM((_ZCHUNK,), jnp.float32),
            pltpu.VMEM((_LBL_ROWS, _LBL_COLS), jnp.int32),
            pltpu.VMEM((_LBL_COLS,), jnp.float32),
            pltpu.SemaphoreType.DMA,
        ],
    )(lbl3)


def _tc_body(x_ref, m_ref, o_ref):
    feat = jnp.where(m_ref[...] != 0, jnp.float32(1.0), x_ref[...])
    o_ref[...] = jnp.broadcast_to(feat, (feat.shape[0], _F_OUT))


def kernel(x, shape, labels):
    del shape
    n = x.shape[0]
    mask2d = _make_mask(labels).reshape(_MASK_N, 1)
    return pl.pallas_call(
        _tc_body,
        grid=(n // _BLK,),
        in_specs=[
            pl.BlockSpec((_BLK, 1), lambda i: (i, 0)),
            pl.BlockSpec((_BLK, 1), lambda i: (i, 0)),
        ],
        out_specs=pl.BlockSpec((_BLK, _F_OUT), lambda i: (i, 0)),
        out_shape=jax.ShapeDtypeStruct((n, _F_OUT), jnp.float32),
        compiler_params=pltpu.CompilerParams(
            dimension_semantics=("arbitrary",),
        ),
    )(x, mask2d)
